# baseline (speedup 1.0000x reference)
import jax
import jax.numpy as jnp
from jax import lax
from jax.experimental import pallas as pl
from jax.experimental.pallas import tpu as pltpu
from jax.experimental.pallas import tpu_sc as plsc

DIM = 32
BATCH = 16384
NUM_WORKERS = 32
B_PER_W = BATCH // NUM_WORKERS
IDX_CHUNK = 128
N_CHUNKS = B_PER_W // IDX_CHUNK
LANES = 16
N_GROUPS = B_PER_W // LANES


def _mf_body(user_hbm, item_hbm, utab_hbm, itab_hbm, out_hbm,
             uidx_v, iidx_v, urows_v, irows_v, out_v, sem):
    wid = lax.axis_index("s") * 2 + lax.axis_index("c")
    base = wid * B_PER_W

    for ch in range(N_CHUNKS):
        off = base + ch * IDX_CHUNK
        pltpu.sync_copy(user_hbm.at[pl.ds(off, IDX_CHUNK)], uidx_v.at[ch])
        pltpu.sync_copy(item_hbm.at[pl.ds(off, IDX_CHUNK)], iidx_v.at[ch])

    copies = []
    for ch in range(N_CHUNKS):
        dst = pl.ds(ch * IDX_CHUNK, IDX_CHUNK)
        copies.append(pltpu.async_copy(utab_hbm.at[uidx_v.at[ch]],
                                       urows_v.at[dst], sem))
        copies.append(pltpu.async_copy(itab_hbm.at[iidx_v.at[ch]],
                                       irows_v.at[dst], sem))
    for c in copies:
        c.wait()

    iota = lax.iota(jnp.int32, LANES)

    def group_body(g, carry):
        rows = g * LANES + iota
        acc = jnp.zeros((LANES,), jnp.float32)
        for d in range(DIM):
            cols = jnp.full((LANES,), d, jnp.int32)
            uc = plsc.load_gather(urows_v, [rows, cols])
            ic = plsc.load_gather(irows_v, [rows, cols])
            acc = acc + uc * ic
        out_v[pl.ds(g * LANES, LANES)] = acc
        return carry

    lax.fori_loop(0, N_GROUPS, group_body, 0)

    pltpu.sync_copy(out_v, out_hbm.at[pl.ds(base, B_PER_W)])


@jax.jit
def _mf(user, item, user_table, item_table):
    mesh = plsc.VectorSubcoreMesh(core_axis_name="c", subcore_axis_name="s")
    return pl.kernel(
        _mf_body,
        out_type=jax.ShapeDtypeStruct((BATCH,), jnp.float32),
        mesh=mesh,
        compiler_params=pltpu.CompilerParams(
            needs_layout_passes=False,
            use_tc_tiling_on_sc=False,
        ),
        scratch_types=[
            pltpu.VMEM((N_CHUNKS, IDX_CHUNK), jnp.int32),
            pltpu.VMEM((N_CHUNKS, IDX_CHUNK), jnp.int32),
            pltpu.VMEM((B_PER_W, DIM), jnp.float32),
            pltpu.VMEM((B_PER_W, DIM), jnp.float32),
            pltpu.VMEM((B_PER_W,), jnp.float32),
            pltpu.SemaphoreType.DMA,
        ],
    )(user, item, user_table, item_table)


def kernel(user, item, user_table, item_table):
    return _mf(user, item, user_table, item_table)
